# baseline (speedup 1.0000x reference)
import functools

import jax
import jax.numpy as jnp
from jax import lax
from jax.experimental import pallas as pl
from jax.experimental.pallas import tpu as pltpu
from jax.experimental.pallas import tpu_sc as plsc

N = 10000
E = 320000
FEAT = 128
HID = 128

P = 10240
NW = 32
K = 64
EW_T = E // NW
NCH = 160
HNCH = NCH // 2
EP_T = NCH * K
RPT = P // 16

_mesh = plsc.VectorSubcoreMesh(core_axis_name="c", subcore_axis_name="s")


def _sigmoid(w):
    return 1.0 / (1.0 + jnp.exp(-w))


@functools.partial(
    pl.kernel,
    mesh=_mesh,
    out_type=jax.ShapeDtypeStruct((2 * P,), jnp.float32),
    scratch_types=[
        pltpu.VMEM((NCH, K), jnp.int32),
        pltpu.VMEM((NCH, K), jnp.float32),
        pltpu.VMEM((RPT,), jnp.float32),
        pltpu.VMEM_SHARED((P,), jnp.float32),
    ],
)
def _deg_kernel(colp, ewp, degp, col_v, val_v, zb, deg_sh):
    c = lax.axis_index("c")
    s = lax.axis_index("s")
    wid = c * 16 + s
    pltpu.sync_copy(colp.at[wid], col_v)
    pltpu.sync_copy(ewp.at[wid], val_v)

    def sig_body(j, carry):
        for f in range(K // 16):
            sl = pl.ds(f * 16, 16)
            val_v[j, sl] = _sigmoid(val_v[j, sl])
        return carry

    lax.fori_loop(0, NCH, sig_body, None)

    def zb_body(t, carry):
        zb[pl.ds(t * 16, 16)] = jnp.zeros((16,), jnp.float32)
        return carry

    lax.fori_loop(0, RPT // 16, zb_body, None)
    pltpu.sync_copy(zb, deg_sh.at[pl.ds(s * RPT, RPT)])
    plsc.subcore_barrier()

    def sc_body(j, carry):
        pltpu.sync_copy(val_v.at[j], deg_sh.at[col_v.at[j]], add=True)
        return carry

    lax.fori_loop(0, NCH, sc_body, None)
    plsc.subcore_barrier()
    pltpu.sync_copy(deg_sh.at[pl.ds(s * RPT, RPT)], zb)
    pltpu.sync_copy(zb, degp.at[pl.ds(c * P + s * RPT, RPT)])


@functools.partial(
    pl.kernel,
    mesh=_mesh,
    out_type=jax.ShapeDtypeStruct((2 * P, HID), jnp.float32),
    scratch_types=[
        pltpu.VMEM((HNCH, K), jnp.int32),
        pltpu.VMEM((HNCH, K), jnp.int32),
        pltpu.VMEM((HNCH, K), jnp.float32),
        pltpu.VMEM((K, HID), jnp.float32),
        pltpu.VMEM((K, HID), jnp.float32),
        pltpu.VMEM_SHARED((P, HID), jnp.float32),
        pltpu.SemaphoreType.DMA,
        pltpu.SemaphoreType.DMA,
    ],
)
def _msg_kernel(rowp, colp, ewp, g, spart,
                row_v, col_v, scl_v, rows_a, rows_b, s_sh, sem_a, sem_b):
    c = lax.axis_index("c")
    s = lax.axis_index("s")
    wid = c * 16 + s

    def z_body(i, carry):
        for f in range(HID // 16):
            rows_a[i, pl.ds(f * 16, 16)] = jnp.zeros((16,), jnp.float32)
        return carry

    lax.fori_loop(0, K, z_body, None)
    for t in range(RPT // K):
        pltpu.sync_copy(rows_a, s_sh.at[pl.ds(s * RPT + t * K, K)])
    plsc.subcore_barrier()

    def _scale(j, buf):
        def row_scale(gq, carry2):
            s16 = scl_v[j, pl.ds(gq * 16, 16)]
            for l in range(16):
                spl = jnp.broadcast_to(s16[l], (16,))
                e = gq * 16 + l
                for f in range(HID // 16):
                    sl = pl.ds(f * 16, 16)
                    buf[e, sl] = buf[e, sl] * spl
            return carry2

        lax.fori_loop(0, K // 16, row_scale, None)

    for ph in range(2):
        pltpu.sync_copy(rowp.at[wid, pl.ds(ph * HNCH, HNCH)], row_v)
        pltpu.sync_copy(colp.at[wid, pl.ds(ph * HNCH, HNCH)], col_v)
        pltpu.sync_copy(ewp.at[wid, pl.ds(ph * HNCH, HNCH)], scl_v)

        def scl_body(j, carry):
            for f in range(K // 16):
                sl = pl.ds(f * 16, 16)
                scl_v[j, sl] = _sigmoid(scl_v[j, sl])
            return carry

        lax.fori_loop(0, HNCH, scl_body, None)

        pltpu.async_copy(g.at[row_v.at[0]], rows_a, sem_a)
        pltpu.make_async_copy(g.at[row_v.at[0]], rows_a, sem_a).wait()

        def pair_body(p, carry):
            j0 = p * 2
            pltpu.async_copy(g.at[row_v.at[j0 + 1]], rows_b, sem_b)
            _scale(j0, rows_a)
            pltpu.make_async_copy(g.at[row_v.at[j0 + 1]], rows_b, sem_b).wait()
            pltpu.sync_copy(rows_a, s_sh.at[col_v.at[j0]], add=True)
            pltpu.async_copy(g.at[row_v.at[j0 + 2]], rows_a, sem_a)
            _scale(j0 + 1, rows_b)
            pltpu.make_async_copy(g.at[row_v.at[j0 + 2]], rows_a, sem_a).wait()
            pltpu.sync_copy(rows_b, s_sh.at[col_v.at[j0 + 1]], add=True)
            return carry

        lax.fori_loop(0, HNCH // 2 - 1, pair_body, None)
        jl = HNCH - 2
        pltpu.async_copy(g.at[row_v.at[jl + 1]], rows_b, sem_b)
        _scale(jl, rows_a)
        pltpu.make_async_copy(g.at[row_v.at[jl + 1]], rows_b, sem_b).wait()
        pltpu.sync_copy(rows_a, s_sh.at[col_v.at[jl]], add=True)
        _scale(jl + 1, rows_b)
        pltpu.sync_copy(rows_b, s_sh.at[col_v.at[jl + 1]], add=True)

    plsc.subcore_barrier()
    for t in range(RPT // K):
        r0 = s * RPT + t * K
        pltpu.sync_copy(s_sh.at[pl.ds(r0, K)], rows_a)
        pltpu.sync_copy(rows_a, spart.at[pl.ds(c * P + r0, K)])


def _mm_body(x_ref, W_ref, u_ref, p0_ref, p1_ref, o_ref):
    W = W_ref[...]
    u0 = u_ref[...]
    v = jnp.dot(u0, W, preferred_element_type=jnp.float32)
    v = v / (jnp.sqrt(jnp.sum(v * v)) + 1e-12)
    u2 = lax.dot_general(v, W, (((1,), (1,)), ((), ())),
                         preferred_element_type=jnp.float32)
    u2 = u2 / (jnp.sqrt(jnp.sum(u2 * u2)) + 1e-12)
    Wv = lax.dot_general(W, v, (((1,), (1,)), ((), ())),
                         preferred_element_type=jnp.float32)
    sigma = jnp.dot(u2, Wv, preferred_element_type=jnp.float32)[0, 0]
    deg = p0_ref[...] + p1_ref[...] + 1.0
    di = lax.rsqrt(deg)[:, None]
    o_ref[...] = di * jnp.dot(x_ref[...], W,
                              preferred_element_type=jnp.float32) / sigma


def _cb_body(s0_ref, s1_ref, g_ref, p0_ref, p1_ref, b_ref, o_ref):
    deg = p0_ref[...] + p1_ref[...] + 1.0
    di = lax.rsqrt(deg)[:, None]
    o_ref[...] = di * (s0_ref[...] + s1_ref[...] + g_ref[...]) + b_ref[...]


_RB = 256


def kernel(x, edge_index, edge_wt, W, b, u):
    row = edge_index[0]
    col = edge_index[1]

    pad = EP_T - EW_T
    spread = (jnp.arange(NW * pad, dtype=jnp.int32) * 97) % N
    spread = spread.reshape(NW, pad)
    rowp = jnp.concatenate([row.reshape(NW, EW_T), spread], axis=1)
    colp = jnp.concatenate([col.reshape(NW, EW_T), spread], axis=1)
    ewp = jnp.concatenate(
        [edge_wt.reshape(NW, EW_T),
         jnp.full((NW, pad), -1e4, jnp.float32)], axis=1)
    rowp = rowp.reshape(NW, NCH, K)
    colp = colp.reshape(NW, NCH, K)
    ewp = ewp.reshape(NW, NCH, K)

    xp = jnp.pad(x, ((0, P - N), (0, 0)))

    degp = _deg_kernel(colp, ewp)

    g = pl.pallas_call(
        _mm_body,
        grid=(P // _RB,),
        in_specs=[
            pl.BlockSpec((_RB, FEAT), lambda i: (i, 0)),
            pl.BlockSpec((FEAT, HID), lambda i: (0, 0)),
            pl.BlockSpec((1, FEAT), lambda i: (0, 0)),
            pl.BlockSpec((_RB,), lambda i: (i,)),
            pl.BlockSpec((_RB,), lambda i: (i + P // _RB,)),
        ],
        out_specs=pl.BlockSpec((_RB, HID), lambda i: (i, 0)),
        out_shape=jax.ShapeDtypeStruct((P, HID), jnp.float32),
    )(xp, W, u.reshape(1, FEAT), degp, degp)

    spart = _msg_kernel(rowp, colp, ewp, g)

    out = pl.pallas_call(
        _cb_body,
        grid=(P // _RB,),
        in_specs=[
            pl.BlockSpec((_RB, HID), lambda i: (i, 0)),
            pl.BlockSpec((_RB, HID), lambda i: (i + P // _RB, 0)),
            pl.BlockSpec((_RB, HID), lambda i: (i, 0)),
            pl.BlockSpec((_RB,), lambda i: (i,)),
            pl.BlockSpec((_RB,), lambda i: (i + P // _RB,)),
            pl.BlockSpec((1, HID), lambda i: (0, 0)),
        ],
        out_specs=pl.BlockSpec((_RB, HID), lambda i: (i, 0)),
        out_shape=jax.ShapeDtypeStruct((P, HID), jnp.float32),
    )(spart, spart, g, degp, degp, b.reshape(1, HID))

    return out[:N]


# speedup vs baseline: 1.1553x; 1.1553x over previous
import functools

import jax
import jax.numpy as jnp
from jax import lax
from jax.experimental import pallas as pl
from jax.experimental.pallas import tpu as pltpu
from jax.experimental.pallas import tpu_sc as plsc

N = 10000
E = 320000
FEAT = 128
HID = 128

P = 10240
NW = 32
K = 128
EW_T = E // NW
NCH = 80
HNCH = NCH // 2
EP_T = NCH * K
RPT = P // 16

_mesh = plsc.VectorSubcoreMesh(core_axis_name="c", subcore_axis_name="s")


def _sigmoid(w):
    return 1.0 / (1.0 + jnp.exp(-w))


@functools.partial(
    pl.kernel,
    mesh=_mesh,
    out_type=jax.ShapeDtypeStruct((2 * P,), jnp.float32),
    scratch_types=[
        pltpu.VMEM((NCH, K), jnp.int32),
        pltpu.VMEM((NCH, K), jnp.float32),
        pltpu.VMEM((RPT,), jnp.float32),
        pltpu.VMEM_SHARED((P,), jnp.float32),
    ],
)
def _deg_kernel(colp, ewp, degp, col_v, val_v, zb, deg_sh):
    c = lax.axis_index("c")
    s = lax.axis_index("s")
    wid = c * 16 + s
    pltpu.sync_copy(colp.at[wid], col_v)
    pltpu.sync_copy(ewp.at[wid], val_v)

    def sig_body(j, carry):
        for f in range(K // 16):
            sl = pl.ds(f * 16, 16)
            val_v[j, sl] = _sigmoid(val_v[j, sl])
        return carry

    lax.fori_loop(0, NCH, sig_body, None)

    def zb_body(t, carry):
        zb[pl.ds(t * 16, 16)] = jnp.zeros((16,), jnp.float32)
        return carry

    lax.fori_loop(0, RPT // 16, zb_body, None)
    pltpu.sync_copy(zb, deg_sh.at[pl.ds(s * RPT, RPT)])
    plsc.subcore_barrier()

    def sc_body(j, carry):
        pltpu.sync_copy(val_v.at[j], deg_sh.at[col_v.at[j]], add=True)
        return carry

    lax.fori_loop(0, NCH, sc_body, None)
    plsc.subcore_barrier()
    pltpu.sync_copy(deg_sh.at[pl.ds(s * RPT, RPT)], zb)
    pltpu.sync_copy(zb, degp.at[pl.ds(c * P + s * RPT, RPT)])


@functools.partial(
    pl.kernel,
    mesh=_mesh,
    out_type=jax.ShapeDtypeStruct((2 * P, HID), jnp.float32),
    scratch_types=[
        pltpu.VMEM((HNCH, K), jnp.int32),
        pltpu.VMEM((HNCH, K), jnp.int32),
        pltpu.VMEM((HNCH, K), jnp.float32),
        pltpu.VMEM((K, HID), jnp.float32),
        pltpu.VMEM((K, HID), jnp.float32),
        pltpu.VMEM_SHARED((P, HID), jnp.float32),
        pltpu.SemaphoreType.DMA,
        pltpu.SemaphoreType.DMA,
    ],
)
def _msg_kernel(rowp, colp, ewp, g, spart,
                row_v, col_v, scl_v, rows_a, rows_b, s_sh, sem_a, sem_b):
    c = lax.axis_index("c")
    s = lax.axis_index("s")
    wid = c * 16 + s

    def z_body(i, carry):
        for f in range(HID // 16):
            rows_a[i, pl.ds(f * 16, 16)] = jnp.zeros((16,), jnp.float32)
        return carry

    lax.fori_loop(0, 128, z_body, None)
    for t in range(RPT // 128):
        pltpu.sync_copy(rows_a.at[pl.ds(0, 128)],
                        s_sh.at[pl.ds(s * RPT + t * 128, 128)])
    plsc.subcore_barrier()

    def _scale(j, buf):
        def row_scale(gq, carry2):
            s16 = scl_v[j, pl.ds(gq * 16, 16)]
            for l in range(16):
                spl = jnp.broadcast_to(s16[l], (16,))
                e = gq * 16 + l
                for f in range(HID // 16):
                    sl = pl.ds(f * 16, 16)
                    buf[e, sl] = buf[e, sl] * spl
            return carry2

        lax.fori_loop(0, K // 16, row_scale, None)

    for ph in range(2):
        pltpu.sync_copy(rowp.at[wid, pl.ds(ph * HNCH, HNCH)], row_v)
        pltpu.sync_copy(colp.at[wid, pl.ds(ph * HNCH, HNCH)], col_v)
        pltpu.sync_copy(ewp.at[wid, pl.ds(ph * HNCH, HNCH)], scl_v)

        def scl_body(j, carry):
            for f in range(K // 16):
                sl = pl.ds(f * 16, 16)
                scl_v[j, sl] = _sigmoid(scl_v[j, sl])
            return carry

        lax.fori_loop(0, HNCH, scl_body, None)

        def pair_body(j2, carry):
            j = j2 * 2
            ca = pltpu.async_copy(g.at[row_v.at[j]], rows_a, sem_a)
            cb = pltpu.async_copy(g.at[row_v.at[j + 1]], rows_b, sem_b)
            ca.wait()
            _scale(j, rows_a)
            pltpu.sync_copy(rows_a, s_sh.at[col_v.at[j]], add=True)
            cb.wait()
            _scale(j + 1, rows_b)
            pltpu.sync_copy(rows_b, s_sh.at[col_v.at[j + 1]], add=True)
            return carry

        lax.fori_loop(0, HNCH // 2, pair_body, None)

    plsc.subcore_barrier()
    for t in range(RPT // 128):
        r0 = s * RPT + t * 128
        pltpu.sync_copy(s_sh.at[pl.ds(r0, 128)], rows_a.at[pl.ds(0, 128)])
        pltpu.sync_copy(rows_a.at[pl.ds(0, 128)], spart.at[pl.ds(c * P + r0, 128)])


def _mm_body(x_ref, W_ref, u_ref, p0_ref, p1_ref, o_ref):
    W = W_ref[...]
    u0 = u_ref[...]
    v = jnp.dot(u0, W, preferred_element_type=jnp.float32)
    v = v / (jnp.sqrt(jnp.sum(v * v)) + 1e-12)
    u2 = lax.dot_general(v, W, (((1,), (1,)), ((), ())),
                         preferred_element_type=jnp.float32)
    u2 = u2 / (jnp.sqrt(jnp.sum(u2 * u2)) + 1e-12)
    Wv = lax.dot_general(W, v, (((1,), (1,)), ((), ())),
                         preferred_element_type=jnp.float32)
    sigma = jnp.dot(u2, Wv, preferred_element_type=jnp.float32)[0, 0]
    deg = p0_ref[...] + p1_ref[...] + 1.0
    di = lax.rsqrt(deg)[:, None]
    o_ref[...] = di * jnp.dot(x_ref[...], W,
                              preferred_element_type=jnp.float32) / sigma


def _cb_body(s0_ref, s1_ref, g_ref, p0_ref, p1_ref, b_ref, o_ref):
    deg = p0_ref[...] + p1_ref[...] + 1.0
    di = lax.rsqrt(deg)[:, None]
    o_ref[...] = di * (s0_ref[...] + s1_ref[...] + g_ref[...]) + b_ref[...]


_RB = 256


def kernel(x, edge_index, edge_wt, W, b, u):
    row = edge_index[0]
    col = edge_index[1]

    pad = EP_T - EW_T
    spread = (jnp.arange(NW * pad, dtype=jnp.int32) * 97) % N
    spread = spread.reshape(NW, pad)
    rowp = jnp.concatenate([row.reshape(NW, EW_T), spread], axis=1)
    colp = jnp.concatenate([col.reshape(NW, EW_T), spread], axis=1)
    ewp = jnp.concatenate(
        [edge_wt.reshape(NW, EW_T),
         jnp.full((NW, pad), -1e4, jnp.float32)], axis=1)
    rowp = rowp.reshape(NW, NCH, K)
    colp = colp.reshape(NW, NCH, K)
    ewp = ewp.reshape(NW, NCH, K)

    xp = jnp.pad(x, ((0, P - N), (0, 0)))

    degp = _deg_kernel(colp, ewp)

    g = pl.pallas_call(
        _mm_body,
        grid=(P // _RB,),
        in_specs=[
            pl.BlockSpec((_RB, FEAT), lambda i: (i, 0)),
            pl.BlockSpec((FEAT, HID), lambda i: (0, 0)),
            pl.BlockSpec((1, FEAT), lambda i: (0, 0)),
            pl.BlockSpec((_RB,), lambda i: (i,)),
            pl.BlockSpec((_RB,), lambda i: (i + P // _RB,)),
        ],
        out_specs=pl.BlockSpec((_RB, HID), lambda i: (i, 0)),
        out_shape=jax.ShapeDtypeStruct((P, HID), jnp.float32),
    )(xp, W, u.reshape(1, FEAT), degp, degp)

    spart = _msg_kernel(rowp, colp, ewp, g)

    out = pl.pallas_call(
        _cb_body,
        grid=(P // _RB,),
        in_specs=[
            pl.BlockSpec((_RB, HID), lambda i: (i, 0)),
            pl.BlockSpec((_RB, HID), lambda i: (i + P // _RB, 0)),
            pl.BlockSpec((_RB, HID), lambda i: (i, 0)),
            pl.BlockSpec((_RB,), lambda i: (i,)),
            pl.BlockSpec((_RB,), lambda i: (i + P // _RB,)),
            pl.BlockSpec((1, HID), lambda i: (0, 0)),
        ],
        out_specs=pl.BlockSpec((_RB, HID), lambda i: (i, 0)),
        out_shape=jax.ShapeDtypeStruct((P, HID), jnp.float32),
    )(spart, spart, g, degp, degp, b.reshape(1, HID))

    return out[:N]


# speedup vs baseline: 1.4266x; 1.2348x over previous
import functools

import jax
import jax.numpy as jnp
from jax import lax
from jax.experimental import pallas as pl
from jax.experimental.pallas import tpu as pltpu
from jax.experimental.pallas import tpu_sc as plsc

N = 10000
E = 320000
FEAT = 128
HID = 128

P = 10240
NW = 32
K = 128
EW_T = E // NW
NCH = 80
HNCH = NCH // 2
EP_T = NCH * K
RPT = P // 16

_mesh = plsc.VectorSubcoreMesh(core_axis_name="c", subcore_axis_name="s")


def _sigmoid(w):
    return 1.0 / (1.0 + jnp.exp(-w))


@functools.partial(
    pl.kernel,
    mesh=_mesh,
    out_type=jax.ShapeDtypeStruct((2 * P,), jnp.float32),
    scratch_types=[
        pltpu.VMEM((NCH, K), jnp.int32),
        pltpu.VMEM((NCH, K), jnp.float32),
        pltpu.VMEM((RPT,), jnp.float32),
        pltpu.VMEM_SHARED((P,), jnp.float32),
    ],
)
def _deg_kernel(colp, ewp, degp, col_v, val_v, zb, deg_sh):
    c = lax.axis_index("c")
    s = lax.axis_index("s")
    wid = c * 16 + s
    pltpu.sync_copy(colp.at[wid], col_v)
    pltpu.sync_copy(ewp.at[wid], val_v)

    def sig_body(j, carry):
        for f in range(K // 16):
            sl = pl.ds(f * 16, 16)
            val_v[j, sl] = _sigmoid(val_v[j, sl])
        return carry

    lax.fori_loop(0, NCH, sig_body, None)

    def zb_body(t, carry):
        zb[pl.ds(t * 16, 16)] = jnp.zeros((16,), jnp.float32)
        return carry

    lax.fori_loop(0, RPT // 16, zb_body, None)
    pltpu.sync_copy(zb, deg_sh.at[pl.ds(s * RPT, RPT)])
    plsc.subcore_barrier()

    def sc_body(j, carry):
        pltpu.sync_copy(val_v.at[j], deg_sh.at[col_v.at[j]], add=True)
        return carry

    lax.fori_loop(0, NCH, sc_body, None)
    plsc.subcore_barrier()
    pltpu.sync_copy(deg_sh.at[pl.ds(s * RPT, RPT)], zb)
    pltpu.sync_copy(zb, degp.at[pl.ds(c * P + s * RPT, RPT)])


@functools.partial(
    pl.kernel,
    mesh=_mesh,
    out_type=jax.ShapeDtypeStruct((2 * P, HID), jnp.float32),
    scratch_types=[
        pltpu.VMEM((HNCH, K), jnp.int32),
        pltpu.VMEM((HNCH, K), jnp.int32),
        pltpu.VMEM((HNCH, K), jnp.float32),
        pltpu.VMEM((K, HID), jnp.float32),
        pltpu.VMEM((K, HID), jnp.float32),
        pltpu.VMEM_SHARED((P, HID), jnp.float32),
        pltpu.SemaphoreType.DMA,
        pltpu.SemaphoreType.DMA,
        pltpu.SemaphoreType.DMA,
        pltpu.SemaphoreType.DMA,
    ],
)
def _msg_kernel(rowp, colp, ewp, g, spart,
                row_v, col_v, scl_v, rows_a, rows_b, s_sh,
                gsem_a, gsem_b, ssem_a, ssem_b):
    c = lax.axis_index("c")
    s = lax.axis_index("s")
    wid = c * 16 + s

    def z_body(i, carry):
        for f in range(HID // 16):
            rows_a[i, pl.ds(f * 16, 16)] = jnp.zeros((16,), jnp.float32)
        return carry

    lax.fori_loop(0, 128, z_body, None)
    for t in range(RPT // 128):
        pltpu.sync_copy(rows_a.at[pl.ds(0, 128)],
                        s_sh.at[pl.ds(s * RPT + t * 128, 128)])
    plsc.subcore_barrier()

    def _scale(j, buf):
        def row_scale(gq, carry2):
            s16 = scl_v[j, pl.ds(gq * 16, 16)]
            for l in range(16):
                spl = jnp.broadcast_to(s16[l], (16,))
                e = gq * 16 + l
                for f in range(HID // 16):
                    sl = pl.ds(f * 16, 16)
                    buf[e, sl] = buf[e, sl] * spl
            return carry2

        lax.fori_loop(0, K // 16, row_scale, None)

    for ph in range(2):
        pltpu.sync_copy(rowp.at[wid, pl.ds(ph * HNCH, HNCH)], row_v)
        pltpu.sync_copy(colp.at[wid, pl.ds(ph * HNCH, HNCH)], col_v)
        pltpu.sync_copy(ewp.at[wid, pl.ds(ph * HNCH, HNCH)], scl_v)

        def scl_body(j, carry):
            for f in range(K // 16):
                sl = pl.ds(f * 16, 16)
                scl_v[j, sl] = _sigmoid(scl_v[j, sl])
            return carry

        lax.fori_loop(0, HNCH, scl_body, None)

        bufs = (rows_a, rows_b)
        gsems = (gsem_a, gsem_b)
        ssems = (ssem_a, ssem_b)

        def gth(jj, b):
            pltpu.async_copy(g.at[row_v.at[jj]], bufs[b], gsems[b])

        def gth_wait(jj, b):
            pltpu.make_async_copy(g.at[row_v.at[jj]], bufs[b],
                                  gsems[b]).wait()

        def sct(jj, b):
            pltpu.async_copy(bufs[b], s_sh.at[col_v.at[jj]], ssems[b],
                             add=True)

        def sct_wait(jj, b):
            pltpu.make_async_copy(bufs[b], s_sh.at[col_v.at[jj]],
                                  ssems[b]).wait()

        gth(0, 0)
        gth_wait(0, 0)
        gth(1, 1)
        _scale(0, rows_a)
        sct(0, 0)

        def pair_body(j2, carry):
            base = 1 + j2 * 2
            for i in range(2):
                jj = base + i
                b = (1 + i) % 2
                gth_wait(jj, b)
                sct_wait(jj - 1, 1 - b)
                gth(jj + 1, 1 - b)
                _scale(jj, bufs[b])
                sct(jj, b)
            return carry

        lax.fori_loop(0, (HNCH - 2) // 2, pair_body, None)

        gth_wait(HNCH - 1, 1)
        _scale(HNCH - 1, rows_b)
        sct(HNCH - 1, 1)
        sct_wait(HNCH - 2, 0)
        sct_wait(HNCH - 1, 1)

    plsc.subcore_barrier()
    for t in range(RPT // 128):
        r0 = s * RPT + t * 128
        pltpu.sync_copy(s_sh.at[pl.ds(r0, 128)], rows_a.at[pl.ds(0, 128)])
        pltpu.sync_copy(rows_a.at[pl.ds(0, 128)], spart.at[pl.ds(c * P + r0, 128)])


def _mm_body(x_ref, W_ref, u_ref, p0_ref, p1_ref, o_ref):
    W = W_ref[...]
    u0 = u_ref[...]
    v = jnp.dot(u0, W, preferred_element_type=jnp.float32)
    v = v / (jnp.sqrt(jnp.sum(v * v)) + 1e-12)
    u2 = lax.dot_general(v, W, (((1,), (1,)), ((), ())),
                         preferred_element_type=jnp.float32)
    u2 = u2 / (jnp.sqrt(jnp.sum(u2 * u2)) + 1e-12)
    Wv = lax.dot_general(W, v, (((1,), (1,)), ((), ())),
                         preferred_element_type=jnp.float32)
    sigma = jnp.dot(u2, Wv, preferred_element_type=jnp.float32)[0, 0]
    deg = p0_ref[...] + p1_ref[...] + 1.0
    di = lax.rsqrt(deg)[:, None]
    o_ref[...] = di * jnp.dot(x_ref[...], W,
                              preferred_element_type=jnp.float32) / sigma


def _cb_body(s0_ref, s1_ref, g_ref, p0_ref, p1_ref, b_ref, o_ref):
    deg = p0_ref[...] + p1_ref[...] + 1.0
    di = lax.rsqrt(deg)[:, None]
    o_ref[...] = di * (s0_ref[...] + s1_ref[...] + g_ref[...]) + b_ref[...]


_RB = 256


def kernel(x, edge_index, edge_wt, W, b, u):
    row = edge_index[0]
    col = edge_index[1]

    pad = EP_T - EW_T
    spread = (jnp.arange(NW * pad, dtype=jnp.int32) * 97) % N
    spread = spread.reshape(NW, pad)
    rowp = jnp.concatenate([row.reshape(NW, EW_T), spread], axis=1)
    colp = jnp.concatenate([col.reshape(NW, EW_T), spread], axis=1)
    ewp = jnp.concatenate(
        [edge_wt.reshape(NW, EW_T),
         jnp.full((NW, pad), -1e4, jnp.float32)], axis=1)
    rowp = rowp.reshape(NW, NCH, K)
    colp = colp.reshape(NW, NCH, K)
    ewp = ewp.reshape(NW, NCH, K)

    xp = jnp.pad(x, ((0, P - N), (0, 0)))

    degp = _deg_kernel(colp, ewp)

    g = pl.pallas_call(
        _mm_body,
        grid=(P // _RB,),
        in_specs=[
            pl.BlockSpec((_RB, FEAT), lambda i: (i, 0)),
            pl.BlockSpec((FEAT, HID), lambda i: (0, 0)),
            pl.BlockSpec((1, FEAT), lambda i: (0, 0)),
            pl.BlockSpec((_RB,), lambda i: (i,)),
            pl.BlockSpec((_RB,), lambda i: (i + P // _RB,)),
        ],
        out_specs=pl.BlockSpec((_RB, HID), lambda i: (i, 0)),
        out_shape=jax.ShapeDtypeStruct((P, HID), jnp.float32),
    )(xp, W, u.reshape(1, FEAT), degp, degp)

    spart = _msg_kernel(rowp, colp, ewp, g)

    out = pl.pallas_call(
        _cb_body,
        grid=(P // _RB,),
        in_specs=[
            pl.BlockSpec((_RB, HID), lambda i: (i, 0)),
            pl.BlockSpec((_RB, HID), lambda i: (i + P // _RB, 0)),
            pl.BlockSpec((_RB, HID), lambda i: (i, 0)),
            pl.BlockSpec((_RB,), lambda i: (i,)),
            pl.BlockSpec((_RB,), lambda i: (i + P // _RB,)),
            pl.BlockSpec((1, HID), lambda i: (0, 0)),
        ],
        out_specs=pl.BlockSpec((_RB, HID), lambda i: (i, 0)),
        out_shape=jax.ShapeDtypeStruct((P, HID), jnp.float32),
    )(spart, spart, g, degp, degp, b.reshape(1, HID))

    return out[:N]


# speedup vs baseline: 1.4966x; 1.0491x over previous
import functools

import jax
import jax.numpy as jnp
from jax import lax
from jax.experimental import pallas as pl
from jax.experimental.pallas import tpu as pltpu
from jax.experimental.pallas import tpu_sc as plsc

N = 10000
E = 320000
FEAT = 128
HID = 128

P = 10240
NW = 32
K = 128
EW_T = E // NW
NCH = 80
HNCH = NCH // 2
EP_T = NCH * K
RPT = P // 16

_mesh = plsc.VectorSubcoreMesh(core_axis_name="c", subcore_axis_name="s")


def _sigmoid(w):
    return 1.0 / (1.0 + jnp.exp(-w))


@functools.partial(
    pl.kernel,
    mesh=_mesh,
    out_type=jax.ShapeDtypeStruct((2 * P,), jnp.float32),
    scratch_types=[
        pltpu.VMEM((NCH, K), jnp.int32),
        pltpu.VMEM((NCH, K), jnp.float32),
        pltpu.VMEM((RPT,), jnp.float32),
        pltpu.VMEM_SHARED((P,), jnp.float32),
    ],
)
def _deg_kernel(colp, ewp, degp, col_v, val_v, zb, deg_sh):
    c = lax.axis_index("c")
    s = lax.axis_index("s")
    wid = c * 16 + s
    pltpu.sync_copy(colp.at[wid], col_v)
    pltpu.sync_copy(ewp.at[wid], val_v)

    def sig_body(j, carry):
        for f in range(K // 16):
            sl = pl.ds(f * 16, 16)
            val_v[j, sl] = _sigmoid(val_v[j, sl])
        return carry

    lax.fori_loop(0, NCH, sig_body, None)

    def zb_body(t, carry):
        zb[pl.ds(t * 16, 16)] = jnp.zeros((16,), jnp.float32)
        return carry

    lax.fori_loop(0, RPT // 16, zb_body, None)
    pltpu.sync_copy(zb, deg_sh.at[pl.ds(s * RPT, RPT)])
    plsc.subcore_barrier()

    def sc_body(j, carry):
        pltpu.sync_copy(val_v.at[j], deg_sh.at[col_v.at[j]], add=True)
        return carry

    lax.fori_loop(0, NCH, sc_body, None)
    plsc.subcore_barrier()
    pltpu.sync_copy(deg_sh.at[pl.ds(s * RPT, RPT)], zb)
    pltpu.sync_copy(zb, degp.at[pl.ds(c * P + s * RPT, RPT)])


@functools.partial(
    pl.kernel,
    mesh=_mesh,
    out_type=jax.ShapeDtypeStruct((2 * P, HID), jnp.float32),
    scratch_types=[
        pltpu.VMEM((HNCH, K), jnp.int32),
        pltpu.VMEM((HNCH, K), jnp.int32),
        pltpu.VMEM((HNCH, K), jnp.float32),
        pltpu.VMEM((K, HID), jnp.float32),
        pltpu.VMEM((K, HID), jnp.float32),
        pltpu.VMEM_SHARED((P, HID), jnp.float32),
        pltpu.SemaphoreType.DMA,
        pltpu.SemaphoreType.DMA,
        pltpu.SemaphoreType.DMA,
        pltpu.SemaphoreType.DMA,
    ],
)
def _msg_kernel(rowp, colp, ewp, g, spart,
                row_v, col_v, scl_v, rows_a, rows_b, s_sh,
                gsem_a, gsem_b, ssem_a, ssem_b):
    c = lax.axis_index("c")
    s = lax.axis_index("s")
    wid = c * 16 + s

    def z_body(i, carry):
        for f in range(HID // 16):
            rows_a[i, pl.ds(f * 16, 16)] = jnp.zeros((16,), jnp.float32)
        return carry

    lax.fori_loop(0, 128, z_body, None)
    for t in range(RPT // 128):
        pltpu.sync_copy(rows_a.at[pl.ds(0, 128)],
                        s_sh.at[pl.ds(s * RPT + t * 128, 128)])
    plsc.subcore_barrier()

    def _scale(j, buf):
        def row_scale(gq, carry2):
            s16 = scl_v[j, pl.ds(gq * 16, 16)]
            for l in range(16):
                spl = jnp.broadcast_to(s16[l], (16,))
                e = gq * 16 + l
                for f in range(HID // 16):
                    sl = pl.ds(f * 16, 16)
                    buf[e, sl] = buf[e, sl] * spl
            return carry2

        lax.fori_loop(0, K // 16, row_scale, None)

    for ph in range(2):
        pltpu.sync_copy(rowp.at[wid, pl.ds(ph * HNCH, HNCH)], row_v)
        pltpu.sync_copy(colp.at[wid, pl.ds(ph * HNCH, HNCH)], col_v)
        pltpu.sync_copy(ewp.at[wid, pl.ds(ph * HNCH, HNCH)], scl_v)

        def scl_body(j, carry):
            for f in range(K // 16):
                sl = pl.ds(f * 16, 16)
                scl_v[j, sl] = _sigmoid(scl_v[j, sl])
            return carry

        lax.fori_loop(0, HNCH, scl_body, None)

        bufs = (rows_a, rows_b)
        gsems = (gsem_a, gsem_b)
        ssems = (ssem_a, ssem_b)

        def gth(jj, b):
            pltpu.async_copy(g.at[row_v.at[jj]], bufs[b], gsems[b])

        def gth_wait(jj, b):
            pltpu.make_async_copy(g.at[row_v.at[jj]], bufs[b],
                                  gsems[b]).wait()

        def sct(jj, b):
            pltpu.async_copy(bufs[b], s_sh.at[col_v.at[jj]], ssems[b],
                             add=True)

        def sct_wait(jj, b):
            pltpu.make_async_copy(bufs[b], s_sh.at[col_v.at[jj]],
                                  ssems[b]).wait()

        gth(0, 0)
        gth_wait(0, 0)
        gth(1, 1)
        _scale(0, rows_a)
        sct(0, 0)

        def pair_body(j2, carry):
            base = 1 + j2 * 2
            for i in range(2):
                jj = base + i
                b = (1 + i) % 2
                gth_wait(jj, b)
                sct_wait(jj - 1, 1 - b)
                gth(jj + 1, 1 - b)
                _scale(jj, bufs[b])
                sct(jj, b)
            return carry

        lax.fori_loop(0, (HNCH - 2) // 2, pair_body, None)

        gth_wait(HNCH - 1, 1)
        _scale(HNCH - 1, rows_b)
        sct(HNCH - 1, 1)
        sct_wait(HNCH - 2, 0)
        sct_wait(HNCH - 1, 1)

    plsc.subcore_barrier()
    for t in range(RPT // 128):
        r0 = s * RPT + t * 128
        pltpu.sync_copy(s_sh.at[pl.ds(r0, 128)], rows_a.at[pl.ds(0, 128)])
        pltpu.sync_copy(rows_a.at[pl.ds(0, 128)], spart.at[pl.ds(c * P + r0, 128)])


def _mm_body(x_ref, W_ref, u_ref, p0_ref, p1_ref, o_ref, isig_ref):
    W = W_ref[...]

    @pl.when(pl.program_id(0) == 0)
    def _():
        u0 = u_ref[...]
        v = jnp.dot(u0, W, preferred_element_type=jnp.float32)
        v = v / (jnp.sqrt(jnp.sum(v * v)) + 1e-12)
        u2 = lax.dot_general(v, W, (((1,), (1,)), ((), ())),
                             preferred_element_type=jnp.float32)
        u2 = u2 / (jnp.sqrt(jnp.sum(u2 * u2)) + 1e-12)
        Wv = lax.dot_general(W, v, (((1,), (1,)), ((), ())),
                             preferred_element_type=jnp.float32)
        sigma = jnp.dot(u2, Wv, preferred_element_type=jnp.float32)[0, 0]
        isig_ref[0] = 1.0 / sigma

    deg = p0_ref[...] + p1_ref[...] + 1.0
    di = lax.rsqrt(deg)[:, None]
    o_ref[...] = di * jnp.dot(x_ref[...], W,
                              preferred_element_type=jnp.float32) * isig_ref[0]


def _cb_body(s0_ref, s1_ref, g_ref, p0_ref, p1_ref, b_ref, o_ref):
    deg = p0_ref[...] + p1_ref[...] + 1.0
    di = lax.rsqrt(deg)[:, None]
    o_ref[...] = di * (s0_ref[...] + s1_ref[...] + g_ref[...]) + b_ref[...]


_RB = 256


def kernel(x, edge_index, edge_wt, W, b, u):
    row = edge_index[0]
    col = edge_index[1]

    pad = EP_T - EW_T
    spread = (jnp.arange(NW * pad, dtype=jnp.int32) * 97) % N
    spread = spread.reshape(NW, pad)
    rowp = jnp.concatenate([row.reshape(NW, EW_T), spread], axis=1)
    colp = jnp.concatenate([col.reshape(NW, EW_T), spread], axis=1)
    ewp = jnp.concatenate(
        [edge_wt.reshape(NW, EW_T),
         jnp.full((NW, pad), -1e4, jnp.float32)], axis=1)
    rowp = rowp.reshape(NW, NCH, K)
    colp = colp.reshape(NW, NCH, K)
    ewp = ewp.reshape(NW, NCH, K)

    degp = _deg_kernel(colp, ewp)

    g = pl.pallas_call(
        _mm_body,
        grid=(P // _RB,),
        in_specs=[
            pl.BlockSpec((_RB, FEAT), lambda i: (i, 0)),
            pl.BlockSpec((FEAT, HID), lambda i: (0, 0)),
            pl.BlockSpec((1, FEAT), lambda i: (0, 0)),
            pl.BlockSpec((_RB,), lambda i: (i,)),
            pl.BlockSpec((_RB,), lambda i: (i + P // _RB,)),
        ],
        out_specs=pl.BlockSpec((_RB, HID), lambda i: (i, 0)),
        out_shape=jax.ShapeDtypeStruct((P, HID), jnp.float32),
        scratch_shapes=[pltpu.SMEM((1,), jnp.float32)],
    )(x, W, u.reshape(1, FEAT), degp, degp)

    spart = _msg_kernel(rowp, colp, ewp, g)

    out = pl.pallas_call(
        _cb_body,
        grid=(P // _RB,),
        in_specs=[
            pl.BlockSpec((_RB, HID), lambda i: (i, 0)),
            pl.BlockSpec((_RB, HID), lambda i: (i + P // _RB, 0)),
            pl.BlockSpec((_RB, HID), lambda i: (i, 0)),
            pl.BlockSpec((_RB,), lambda i: (i,)),
            pl.BlockSpec((_RB,), lambda i: (i + P // _RB,)),
            pl.BlockSpec((1, HID), lambda i: (0, 0)),
        ],
        out_specs=pl.BlockSpec((_RB, HID), lambda i: (i, 0)),
        out_shape=jax.ShapeDtypeStruct((P, HID), jnp.float32),
    )(spart, spart, g, degp, degp, b.reshape(1, HID))

    return out[:N]


# speedup vs baseline: 1.5295x; 1.0220x over previous
import functools

import jax
import jax.numpy as jnp
from jax import lax
from jax.experimental import pallas as pl
from jax.experimental.pallas import tpu as pltpu
from jax.experimental.pallas import tpu_sc as plsc

N = 10000
E = 320000
FEAT = 128
HID = 128

P = 10240
NW = 32
K = 128
EW_T = E // NW
NCH = 80
HNCH = NCH // 2
EP_T = NCH * K
RPT = P // 16

_mesh = plsc.VectorSubcoreMesh(core_axis_name="c", subcore_axis_name="s")


def _sigmoid(w):
    return 1.0 / (1.0 + jnp.exp(-w))


@functools.partial(
    pl.kernel,
    mesh=_mesh,
    out_type=jax.ShapeDtypeStruct((2 * P,), jnp.float32),
    scratch_types=[
        pltpu.VMEM((NCH, K), jnp.int32),
        pltpu.VMEM((NCH, K), jnp.float32),
        pltpu.VMEM((RPT,), jnp.float32),
        pltpu.VMEM_SHARED((P,), jnp.float32),
    ],
)
def _deg_kernel(colp, ewp, degp, col_v, val_v, zb, deg_sh):
    c = lax.axis_index("c")
    s = lax.axis_index("s")
    wid = c * 16 + s
    pltpu.sync_copy(colp.at[wid], col_v)
    pltpu.sync_copy(ewp.at[wid], val_v)

    def sig_body(j, carry):
        for f in range(K // 16):
            sl = pl.ds(f * 16, 16)
            val_v[j, sl] = _sigmoid(val_v[j, sl])
        return carry

    lax.fori_loop(0, NCH, sig_body, None)

    def zb_body(t, carry):
        zb[pl.ds(t * 16, 16)] = jnp.zeros((16,), jnp.float32)
        return carry

    lax.fori_loop(0, RPT // 16, zb_body, None)
    pltpu.sync_copy(zb, deg_sh.at[pl.ds(s * RPT, RPT)])
    plsc.subcore_barrier()

    def sc_body(j, carry):
        pltpu.sync_copy(val_v.at[j], deg_sh.at[col_v.at[j]], add=True)
        return carry

    lax.fori_loop(0, NCH, sc_body, None)
    plsc.subcore_barrier()
    pltpu.sync_copy(deg_sh.at[pl.ds(s * RPT, RPT)], zb)
    pltpu.sync_copy(zb, degp.at[pl.ds(c * P + s * RPT, RPT)])


@functools.partial(
    pl.kernel,
    mesh=_mesh,
    out_type=jax.ShapeDtypeStruct((2 * P, HID), jnp.float32),
    scratch_types=[
        pltpu.VMEM((HNCH, K), jnp.int32),
        pltpu.VMEM((HNCH, K), jnp.int32),
        pltpu.VMEM((HNCH, K), jnp.float32),
        pltpu.VMEM((K, HID), jnp.float32),
        pltpu.VMEM((K, HID), jnp.float32),
        pltpu.VMEM_SHARED((P, HID), jnp.float32),
        pltpu.SemaphoreType.DMA,
        pltpu.SemaphoreType.DMA,
        pltpu.SemaphoreType.DMA,
        pltpu.SemaphoreType.DMA,
    ],
)
def _msg_kernel(rowp, colp, ewp, g, spart,
                row_v, col_v, scl_v, rows_a, rows_b, s_sh,
                gsem_a, gsem_b, ssem_a, ssem_b):
    c = lax.axis_index("c")
    s = lax.axis_index("s")
    wid = c * 16 + s

    def z_body(i, carry):
        for f in range(HID // 16):
            rows_a[i, pl.ds(f * 16, 16)] = jnp.zeros((16,), jnp.float32)
        return carry

    lax.fori_loop(0, 128, z_body, None)
    for t in range(RPT // 128):
        pltpu.sync_copy(rows_a.at[pl.ds(0, 128)],
                        s_sh.at[pl.ds(s * RPT + t * 128, 128)])
    plsc.subcore_barrier()

    def _scale(j, buf):
        def row_scale(gq, carry2):
            s16 = scl_v[j, pl.ds(gq * 16, 16)]
            for l in range(16):
                spl = jnp.broadcast_to(s16[l], (16,))
                e = gq * 16 + l
                for f in range(HID // 16):
                    sl = pl.ds(f * 16, 16)
                    buf[e, sl] = buf[e, sl] * spl
            return carry2

        lax.fori_loop(0, K // 16, row_scale, None)

    for ph in range(2):
        pltpu.sync_copy(rowp.at[wid, pl.ds(ph * HNCH, HNCH)], row_v)
        pltpu.sync_copy(colp.at[wid, pl.ds(ph * HNCH, HNCH)], col_v)
        pltpu.sync_copy(ewp.at[wid, pl.ds(ph * HNCH, HNCH)], scl_v)

        def scl_body(j, carry):
            for f in range(K // 16):
                sl = pl.ds(f * 16, 16)
                scl_v[j, sl] = _sigmoid(scl_v[j, sl])
            return carry

        lax.fori_loop(0, HNCH, scl_body, None)

        bufs = (rows_a, rows_b)
        gsems = (gsem_a, gsem_b)
        ssems = (ssem_a, ssem_b)

        def gth(jj, b):
            pltpu.async_copy(g.at[row_v.at[jj]], bufs[b], gsems[b])

        def gth_wait(jj, b):
            pltpu.make_async_copy(g.at[row_v.at[jj]], bufs[b],
                                  gsems[b]).wait()

        def sct(jj, b):
            pltpu.async_copy(bufs[b], s_sh.at[col_v.at[jj]], ssems[b],
                             add=True)

        def sct_wait(jj, b):
            pltpu.make_async_copy(bufs[b], s_sh.at[col_v.at[jj]],
                                  ssems[b]).wait()

        gth(0, 0)
        gth_wait(0, 0)
        gth(1, 1)
        _scale(0, rows_a)
        sct(0, 0)

        def pair_body(j2, carry):
            base = 1 + j2 * 2
            for i in range(2):
                jj = base + i
                b = (1 + i) % 2
                gth_wait(jj, b)
                sct_wait(jj - 1, 1 - b)
                gth(jj + 1, 1 - b)
                _scale(jj, bufs[b])
                sct(jj, b)
            return carry

        lax.fori_loop(0, (HNCH - 2) // 2, pair_body, None)

        gth_wait(HNCH - 1, 1)
        _scale(HNCH - 1, rows_b)
        sct(HNCH - 1, 1)
        sct_wait(HNCH - 2, 0)
        sct_wait(HNCH - 1, 1)

    plsc.subcore_barrier()
    for t in range(RPT // 128):
        r0 = s * RPT + t * 128
        pltpu.sync_copy(s_sh.at[pl.ds(r0, 128)], rows_a.at[pl.ds(0, 128)])
        pltpu.sync_copy(rows_a.at[pl.ds(0, 128)], spart.at[pl.ds(c * P + r0, 128)])


def _mm_body(x_ref, W_ref, u_ref, p0_ref, p1_ref, o_ref, isig_ref):
    W = W_ref[...]

    @pl.when(pl.program_id(0) == 0)
    def _():
        u0 = u_ref[...]
        v = jnp.dot(u0, W, preferred_element_type=jnp.float32)
        v = v / (jnp.sqrt(jnp.sum(v * v)) + 1e-12)
        u2 = lax.dot_general(v, W, (((1,), (1,)), ((), ())),
                             preferred_element_type=jnp.float32)
        u2 = u2 / (jnp.sqrt(jnp.sum(u2 * u2)) + 1e-12)
        Wv = lax.dot_general(W, v, (((1,), (1,)), ((), ())),
                             preferred_element_type=jnp.float32)
        sigma = jnp.dot(u2, Wv, preferred_element_type=jnp.float32)[0, 0]
        isig_ref[0] = 1.0 / sigma

    deg = p0_ref[...] + p1_ref[...] + 1.0
    di = lax.rsqrt(deg)[:, None]
    o_ref[...] = di * jnp.dot(x_ref[...], W,
                              preferred_element_type=jnp.float32) * isig_ref[0]


def _cb_body(s0_ref, s1_ref, g_ref, p0_ref, p1_ref, b_ref, o_ref):
    deg = p0_ref[...] + p1_ref[...] + 1.0
    di = lax.rsqrt(deg)[:, None]
    o_ref[...] = di * (s0_ref[...] + s1_ref[...] + g_ref[...]) + b_ref[...]


_RB = 256


def kernel(x, edge_index, edge_wt, W, b, u):
    row = edge_index[0]
    col = edge_index[1]

    pad = EP_T - EW_T
    spread = (jnp.arange(NW * pad, dtype=jnp.int32) * 97) % N
    spread = spread.reshape(NW, pad)
    rowp = jnp.concatenate([row.reshape(NW, EW_T), spread], axis=1)
    colp = jnp.concatenate([col.reshape(NW, EW_T), spread], axis=1)
    ewp = jnp.concatenate(
        [edge_wt.reshape(NW, EW_T),
         jnp.full((NW, pad), -1e4, jnp.float32)], axis=1)
    rowp = rowp.reshape(NW, NCH, K)
    colp = colp.reshape(NW, NCH, K)
    ewp = ewp.reshape(NW, NCH, K)

    degp = _deg_kernel(colp, ewp)

    g = pl.pallas_call(
        _mm_body,
        grid=(P // _RB,),
        in_specs=[
            pl.BlockSpec((_RB, FEAT), lambda i: (i, 0)),
            pl.BlockSpec((FEAT, HID), lambda i: (0, 0)),
            pl.BlockSpec((1, FEAT), lambda i: (0, 0)),
            pl.BlockSpec((_RB,), lambda i: (i,)),
            pl.BlockSpec((_RB,), lambda i: (i + P // _RB,)),
        ],
        out_specs=pl.BlockSpec((_RB, HID), lambda i: (i, 0)),
        out_shape=jax.ShapeDtypeStruct((P, HID), jnp.float32),
        scratch_shapes=[pltpu.SMEM((1,), jnp.float32)],
    )(x, W, u.reshape(1, FEAT), degp, degp)

    spart = _msg_kernel(rowp, colp, ewp, g)

    out = pl.pallas_call(
        _cb_body,
        grid=(P // _RB,),
        in_specs=[
            pl.BlockSpec((_RB, HID), lambda i: (i, 0)),
            pl.BlockSpec((_RB, HID), lambda i: (i + P // _RB, 0)),
            pl.BlockSpec((_RB, HID), lambda i: (i, 0)),
            pl.BlockSpec((_RB,), lambda i: (i,)),
            pl.BlockSpec((_RB,), lambda i: (i + P // _RB,)),
            pl.BlockSpec((1, HID), lambda i: (0, 0)),
        ],
        out_specs=pl.BlockSpec((_RB, HID), lambda i: (i, 0)),
        out_shape=jax.ShapeDtypeStruct((N, HID), jnp.float32),
    )(spart, spart, g, degp, degp, b.reshape(1, HID))

    return out


# speedup vs baseline: 1.5868x; 1.0375x over previous
import functools

import jax
import jax.numpy as jnp
from jax import lax
from jax.experimental import pallas as pl
from jax.experimental.pallas import tpu as pltpu
from jax.experimental.pallas import tpu_sc as plsc

N = 10000
E = 320000
FEAT = 128
HID = 128

P = 10240
NW = 32
K = 128
EW_T = E // NW
NCH = 80
HNCH = NCH // 2
EP_T = NCH * K
RPT = P // 16

_mesh = plsc.VectorSubcoreMesh(core_axis_name="c", subcore_axis_name="s")


def _sigmoid(w):
    return 1.0 / (1.0 + jnp.exp(-w))


@functools.partial(
    pl.kernel,
    mesh=_mesh,
    out_type=jax.ShapeDtypeStruct((2 * P,), jnp.float32),
    scratch_types=[
        pltpu.VMEM((NCH, K), jnp.int32),
        pltpu.VMEM((NCH, K), jnp.float32),
        pltpu.VMEM((RPT,), jnp.float32),
        pltpu.VMEM_SHARED((P,), jnp.float32),
    ],
)
def _deg_kernel(eip, ewp, degp, col_v, val_v, zb, deg_sh):
    c = lax.axis_index("c")
    s = lax.axis_index("s")
    wid = c * 16 + s
    pltpu.sync_copy(eip.at[1, wid], col_v)
    pltpu.sync_copy(ewp.at[wid], val_v)

    def sig_body(j, carry):
        for f in range(K // 16):
            sl = pl.ds(f * 16, 16)
            val_v[j, sl] = _sigmoid(val_v[j, sl])
        return carry

    lax.fori_loop(0, NCH, sig_body, None)

    def zb_body(t, carry):
        zb[pl.ds(t * 16, 16)] = jnp.zeros((16,), jnp.float32)
        return carry

    lax.fori_loop(0, RPT // 16, zb_body, None)
    pltpu.sync_copy(zb, deg_sh.at[pl.ds(s * RPT, RPT)])
    plsc.subcore_barrier()

    def sc_body(j, carry):
        pltpu.sync_copy(val_v.at[j], deg_sh.at[col_v.at[j]], add=True)
        return carry

    lax.fori_loop(0, NCH, sc_body, None)
    plsc.subcore_barrier()
    pltpu.sync_copy(deg_sh.at[pl.ds(s * RPT, RPT)], zb)
    pltpu.sync_copy(zb, degp.at[pl.ds(c * P + s * RPT, RPT)])


@functools.partial(
    pl.kernel,
    mesh=_mesh,
    out_type=jax.ShapeDtypeStruct((2 * P, HID), jnp.float32),
    scratch_types=[
        pltpu.VMEM((HNCH, K), jnp.int32),
        pltpu.VMEM((HNCH, K), jnp.int32),
        pltpu.VMEM((HNCH, K), jnp.float32),
        pltpu.VMEM((K, HID), jnp.float32),
        pltpu.VMEM((K, HID), jnp.float32),
        pltpu.VMEM_SHARED((P, HID), jnp.float32),
        pltpu.SemaphoreType.DMA,
        pltpu.SemaphoreType.DMA,
        pltpu.SemaphoreType.DMA,
        pltpu.SemaphoreType.DMA,
    ],
)
def _msg_kernel(eip, ewp, g, spart,
                row_v, col_v, scl_v, rows_a, rows_b, s_sh,
                gsem_a, gsem_b, ssem_a, ssem_b):
    c = lax.axis_index("c")
    s = lax.axis_index("s")
    wid = c * 16 + s

    def z_body(i, carry):
        for f in range(HID // 16):
            rows_a[i, pl.ds(f * 16, 16)] = jnp.zeros((16,), jnp.float32)
        return carry

    lax.fori_loop(0, 128, z_body, None)
    for t in range(RPT // 128):
        pltpu.sync_copy(rows_a.at[pl.ds(0, 128)],
                        s_sh.at[pl.ds(s * RPT + t * 128, 128)])
    plsc.subcore_barrier()

    def _scale(j, buf):
        def row_scale(gq, carry2):
            s16 = scl_v[j, pl.ds(gq * 16, 16)]
            for l in range(16):
                spl = jnp.broadcast_to(s16[l], (16,))
                e = gq * 16 + l
                for f in range(HID // 16):
                    sl = pl.ds(f * 16, 16)
                    buf[e, sl] = buf[e, sl] * spl
            return carry2

        lax.fori_loop(0, K // 16, row_scale, None)

    for ph in range(2):
        pltpu.sync_copy(eip.at[0, wid, pl.ds(ph * HNCH, HNCH)], row_v)
        pltpu.sync_copy(eip.at[1, wid, pl.ds(ph * HNCH, HNCH)], col_v)
        pltpu.sync_copy(ewp.at[wid, pl.ds(ph * HNCH, HNCH)], scl_v)

        def scl_body(j, carry):
            for f in range(K // 16):
                sl = pl.ds(f * 16, 16)
                scl_v[j, sl] = _sigmoid(scl_v[j, sl])
            return carry

        lax.fori_loop(0, HNCH, scl_body, None)

        bufs = (rows_a, rows_b)
        gsems = (gsem_a, gsem_b)
        ssems = (ssem_a, ssem_b)

        def gth(jj, b):
            pltpu.async_copy(g.at[row_v.at[jj]], bufs[b], gsems[b])

        def gth_wait(jj, b):
            pltpu.make_async_copy(g.at[row_v.at[jj]], bufs[b],
                                  gsems[b]).wait()

        def sct(jj, b):
            pltpu.async_copy(bufs[b], s_sh.at[col_v.at[jj]], ssems[b],
                             add=True)

        def sct_wait(jj, b):
            pltpu.make_async_copy(bufs[b], s_sh.at[col_v.at[jj]],
                                  ssems[b]).wait()

        gth(0, 0)
        gth_wait(0, 0)
        gth(1, 1)
        _scale(0, rows_a)
        sct(0, 0)

        def pair_body(j2, carry):
            base = 1 + j2 * 2
            for i in range(2):
                jj = base + i
                b = (1 + i) % 2
                gth_wait(jj, b)
                sct_wait(jj - 1, 1 - b)
                gth(jj + 1, 1 - b)
                _scale(jj, bufs[b])
                sct(jj, b)
            return carry

        lax.fori_loop(0, (HNCH - 2) // 2, pair_body, None)

        gth_wait(HNCH - 1, 1)
        _scale(HNCH - 1, rows_b)
        sct(HNCH - 1, 1)
        sct_wait(HNCH - 2, 0)
        sct_wait(HNCH - 1, 1)

    plsc.subcore_barrier()
    for t in range(RPT // 128):
        r0 = s * RPT + t * 128
        pltpu.sync_copy(s_sh.at[pl.ds(r0, 128)], rows_a.at[pl.ds(0, 128)])
        pltpu.sync_copy(rows_a.at[pl.ds(0, 128)], spart.at[pl.ds(c * P + r0, 128)])


def _mm_body(x_ref, W_ref, u_ref, p0_ref, p1_ref, o_ref, isig_ref):
    W = W_ref[...]

    @pl.when(pl.program_id(0) == 0)
    def _():
        u0 = u_ref[...]
        v = jnp.dot(u0, W, preferred_element_type=jnp.float32)
        v = v / (jnp.sqrt(jnp.sum(v * v)) + 1e-12)
        u2 = lax.dot_general(v, W, (((1,), (1,)), ((), ())),
                             preferred_element_type=jnp.float32)
        u2 = u2 / (jnp.sqrt(jnp.sum(u2 * u2)) + 1e-12)
        Wv = lax.dot_general(W, v, (((1,), (1,)), ((), ())),
                             preferred_element_type=jnp.float32)
        sigma = jnp.dot(u2, Wv, preferred_element_type=jnp.float32)[0, 0]
        isig_ref[0] = 1.0 / sigma

    deg = p0_ref[...] + p1_ref[...] + 1.0
    di = lax.rsqrt(deg)[:, None]
    o_ref[...] = di * jnp.dot(x_ref[...], W,
                              preferred_element_type=jnp.float32) * isig_ref[0]


def _cb_body(s0_ref, s1_ref, g_ref, p0_ref, p1_ref, b_ref, o_ref):
    deg = p0_ref[...] + p1_ref[...] + 1.0
    di = lax.rsqrt(deg)[:, None]
    o_ref[...] = di * (s0_ref[...] + s1_ref[...] + g_ref[...]) + b_ref[...]


_RB = 256


def kernel(x, edge_index, edge_wt, W, b, u):
    npad = NW * (EP_T - EW_T)
    spread = (jnp.arange(npad, dtype=jnp.int32) * 97) % N
    eip = jnp.concatenate(
        [edge_index.astype(jnp.int32),
         jnp.broadcast_to(spread, (2, npad))], axis=1)
    eip = eip.reshape(2, NW, NCH, K)
    ewp = jnp.concatenate(
        [edge_wt, jnp.full((npad,), -1e4, jnp.float32)])
    ewp = ewp.reshape(NW, NCH, K)

    degp = _deg_kernel(eip, ewp)

    g = pl.pallas_call(
        _mm_body,
        grid=(P // _RB,),
        in_specs=[
            pl.BlockSpec((_RB, FEAT), lambda i: (i, 0)),
            pl.BlockSpec((FEAT, HID), lambda i: (0, 0)),
            pl.BlockSpec((1, FEAT), lambda i: (0, 0)),
            pl.BlockSpec((_RB,), lambda i: (i,)),
            pl.BlockSpec((_RB,), lambda i: (i + P // _RB,)),
        ],
        out_specs=pl.BlockSpec((_RB, HID), lambda i: (i, 0)),
        out_shape=jax.ShapeDtypeStruct((P, HID), jnp.float32),
        scratch_shapes=[pltpu.SMEM((1,), jnp.float32)],
    )(x, W, u.reshape(1, FEAT), degp, degp)

    spart = _msg_kernel(eip, ewp, g)

    out = pl.pallas_call(
        _cb_body,
        grid=(P // _RB,),
        in_specs=[
            pl.BlockSpec((_RB, HID), lambda i: (i, 0)),
            pl.BlockSpec((_RB, HID), lambda i: (i + P // _RB, 0)),
            pl.BlockSpec((_RB, HID), lambda i: (i, 0)),
            pl.BlockSpec((_RB,), lambda i: (i,)),
            pl.BlockSpec((_RB,), lambda i: (i + P // _RB,)),
            pl.BlockSpec((1, HID), lambda i: (0, 0)),
        ],
        out_specs=pl.BlockSpec((_RB, HID), lambda i: (i, 0)),
        out_shape=jax.ShapeDtypeStruct((N, HID), jnp.float32),
    )(spart, spart, g, degp, degp, b.reshape(1, HID))

    return out
